# baseline (speedup 1.0000x reference)
import functools

import jax
import jax.numpy as jnp
from jax import lax
from jax.experimental import pallas as pl
from jax.experimental.pallas import tpu as pltpu
from jax.experimental.pallas import tpu_sc as plsc

B = 4096
C = 20
V = 100000
D = 64
N_NEGS = 20
K_REAL = C + C * N_NEGS
K = 432
NW = 32
BPW = B // NW
CHUNKS = ((0, 128), (128, 128), (256, 128), (384, 48))
RED_BS = 512


def _make_sc_dots(b, d, k, bpw, chunks, interpret=False):
    nw = b // bpw
    nc, ns = 2, 16
    assert nw == nc * ns
    mesh = plsc.VectorSubcoreMesh(
        core_axis_name="c", subcore_axis_name="s",
        num_cores=nc, num_subcores=ns)
    nt = d // 16

    kn = 400
    ko = 20

    @functools.partial(
        pl.kernel,
        out_type=jax.ShapeDtypeStruct((b, k), jnp.float32),
        mesh=mesh,
        interpret=interpret,
        compiler_params=pltpu.CompilerParams(
            needs_layout_passes=False, use_tc_tiling_on_sc=False),
        scratch_types=[
            pltpu.VMEM((bpw,), jnp.int32),
            pltpu.VMEM((bpw, d), jnp.float32),
            pltpu.VMEM((bpw, ko), jnp.int32),
            pltpu.VMEM((bpw, kn), jnp.int32),
            pltpu.VMEM((2, k, d), jnp.bfloat16),
            pltpu.VMEM((k + 16,), jnp.float32),
            pltpu.SemaphoreType.DMA,
            pltpu.SemaphoreType.DMA,
            pltpu.SemaphoreType.DMA,
        ],
    )
    def sc_dots(wi_hbm, wo_hbm, oit_hbm, nit_hbm, iit_hbm, out_hbm,
                iidx_v, ivec_v, oidx_v, nidx_v, rows_v, dots_v,
                sem_a, sem_b, sem_i):
        wid = lax.axis_index("s") * nc + lax.axis_index("c")
        base = wid * bpw
        pltpu.sync_copy(iit_hbm.at[pl.ds(base, bpw)], iidx_v)
        pltpu.async_copy(wi_hbm.at[iidx_v], ivec_v, sem_i).wait()
        pltpu.sync_copy(oit_hbm.at[pl.ds(base, bpw), :], oidx_v)
        pltpu.sync_copy(nit_hbm.at[pl.ds(base, bpw), :], nidx_v)

        def fire(bb, buf, sem):
            pltpu.async_copy(
                wo_hbm.at[oidx_v.at[bb, :]], rows_v.at[buf, pl.ds(0, ko)], sem)
            pltpu.async_copy(
                wo_hbm.at[nidx_v.at[bb, :]], rows_v.at[buf, pl.ds(ko, kn)], sem)

        def drain(bb, buf, sem):
            pltpu.make_async_copy(
                wo_hbm.at[oidx_v.at[bb, :]], rows_v.at[buf, pl.ds(0, ko)], sem).wait()
            pltpu.make_async_copy(
                wo_hbm.at[nidx_v.at[bb, :]], rows_v.at[buf, pl.ds(ko, kn)], sem).wait()

        fire(0, 0, sem_a)

        _LANE = lax.iota(jnp.int32, 16)
        _LAST = _LANE == 15
        _PRM = [(jnp.full((16,), 32 * t, jnp.int32) + 2 * _LANE + p)
                for t in range(nt // 2) for p in (0, 1)]

        def compute(bb, buf):
            bbv = jnp.full((16,), bb, jnp.int32)
            iv = [plsc.load_gather(ivec_v, [bbv, pidx]) for pidx in _PRM]

            @pl.loop(0, k // 16)
            def _grp(g):
                r0 = g * 16
                for j in range(16):
                    acc = None
                    for t in range(nt // 2):
                        v = rows_v[buf, r0 + j, pl.ds(32 * t, 32)]
                        e, o = plsc.unpack(v, format=plsc.PackFormat.INTERLEAVED)
                        term = e * iv[2 * t] + o * iv[2 * t + 1]
                        acc = term if acc is None else acc + term
                    plsc.store_compressed(dots_v.at[pl.ds(r0 + j, 16)],
                                          plsc.cumsum(acc), mask=_LAST)

            pltpu.sync_copy(dots_v.at[pl.ds(0, k)], out_hbm.at[base + bb])

        @pl.loop(0, bpw // 2)
        def _pair(q):
            b0 = 2 * q
            fire(b0 + 1, 1, sem_b)
            drain(b0, 0, sem_a)
            compute(b0, 0)

            @pl.when(b0 + 2 < bpw)
            def _():
                fire(b0 + 2, 0, sem_a)

            drain(b0 + 1, 1, sem_b)
            compute(b0 + 1, 1)

    return sc_dots


def _make_reduce(b, k, bs, c, k_real, interpret=False):
    grid = (b // bs,)

    def red(dots_ref, out_ref, acc_ref):
        x = dots_ref[...]
        col = lax.broadcasted_iota(jnp.int32, (bs, k), 1)
        z = jnp.where(col < c, x, -x)
        l = jnp.log(jax.nn.sigmoid(z))
        l = jnp.where(col < k_real, l, 0.0)

        @pl.when(pl.program_id(0) == 0)
        def _():
            acc_ref[0] = 0.0

        acc_ref[0] = acc_ref[0] + jnp.sum(l)

        @pl.when(pl.program_id(0) == grid[0] - 1)
        def _():
            out_ref[0] = -acc_ref[0] / b

    return pl.pallas_call(
        red,
        grid=grid,
        in_specs=[pl.BlockSpec((bs, k), lambda i: (i, 0))],
        out_specs=pl.BlockSpec(memory_space=pltpu.SMEM),
        out_shape=jax.ShapeDtypeStruct((1,), jnp.float32),
        scratch_shapes=[pltpu.SMEM((1,), jnp.float32)],
        interpret=interpret,
    )


_sc_dots = None
_reduce = None


def kernel(iitem, oitems, nitems, W_i, W_o):
    global _sc_dots, _reduce
    if _sc_dots is None:
        _sc_dots = _make_sc_dots(B, D, K, BPW, CHUNKS)
        _reduce = _make_reduce(B, K, RED_BS, C, K_REAL)
    if oitems.dtype != jnp.int32:
        iitem = iitem.astype(jnp.int32)
        oitems = oitems.astype(jnp.int32)
        nitems = nitems.astype(jnp.int32)
    dots = _sc_dots(W_i, W_o.astype(jnp.bfloat16), oitems, nitems, iitem)
    return _reduce(dots)[0]


# speedup vs baseline: 2.7731x; 2.7731x over previous
import functools

import jax
import jax.numpy as jnp
from jax import lax
from jax.experimental import pallas as pl
from jax.experimental.pallas import tpu as pltpu
from jax.experimental.pallas import tpu_sc as plsc

B = 4096
C = 20
V = 100000
D = 64
N_NEGS = 20
K_REAL = C + C * N_NEGS
K = 432
NW = 32
BPW = B // NW
CHUNKS = ((0, 128), (128, 128), (256, 128), (384, 48))
RED_BS = 512


def _make_sc_dots(b, d, k, bpw, chunks, interpret=False):
    nw = b // bpw
    nc, ns = 2, 16
    assert nw == nc * ns
    mesh = plsc.VectorSubcoreMesh(
        core_axis_name="c", subcore_axis_name="s",
        num_cores=nc, num_subcores=ns)
    nt = d // 16

    kn = 400
    ko = 20

    @functools.partial(
        pl.kernel,
        out_type=jax.ShapeDtypeStruct((b, k), jnp.float32),
        mesh=mesh,
        interpret=interpret,
        compiler_params=pltpu.CompilerParams(
            needs_layout_passes=False, use_tc_tiling_on_sc=False),
        scratch_types=[
            pltpu.VMEM((bpw,), jnp.int32),
            pltpu.VMEM((bpw, d), jnp.float32),
            pltpu.VMEM((bpw, ko), jnp.int32),
            pltpu.VMEM((bpw, kn), jnp.int32),
            pltpu.VMEM((2, k, d), jnp.bfloat16),
            pltpu.VMEM((k,), jnp.float32),
            pltpu.SemaphoreType.DMA,
            pltpu.SemaphoreType.DMA,
            pltpu.SemaphoreType.DMA,
        ],
    )
    def sc_dots(wi_hbm, wo_hbm, oit_hbm, nit_hbm, iit_hbm, out_hbm,
                iidx_v, ivec_v, oidx_v, nidx_v, rows_v, dots_v,
                sem_a, sem_b, sem_i):
        wid = lax.axis_index("s") * nc + lax.axis_index("c")
        base = wid * bpw
        pltpu.sync_copy(iit_hbm.at[pl.ds(base, bpw)], iidx_v)
        pltpu.async_copy(wi_hbm.at[iidx_v], ivec_v, sem_i).wait()
        pltpu.sync_copy(oit_hbm.at[pl.ds(base, bpw), :], oidx_v)
        pltpu.sync_copy(nit_hbm.at[pl.ds(base, bpw), :], nidx_v)

        def fire(bb, buf, sem):
            pltpu.async_copy(
                wo_hbm.at[oidx_v.at[bb, :]], rows_v.at[buf, pl.ds(0, ko)], sem)
            pltpu.async_copy(
                wo_hbm.at[nidx_v.at[bb, :]], rows_v.at[buf, pl.ds(ko, kn)], sem)

        def drain(bb, buf, sem):
            pltpu.make_async_copy(
                wo_hbm.at[oidx_v.at[bb, :]], rows_v.at[buf, pl.ds(0, ko)], sem).wait()
            pltpu.make_async_copy(
                wo_hbm.at[nidx_v.at[bb, :]], rows_v.at[buf, pl.ds(ko, kn)], sem).wait()

        fire(0, 0, sem_a)

        _LANE = lax.iota(jnp.int32, 16)
        _PRM = [(jnp.full((16,), 32 * t, jnp.int32) + 2 * _LANE + p)
                for t in range(nt // 2) for p in (0, 1)]

        def compute(bb, buf):
            bbv = jnp.full((16,), bb, jnp.int32)
            iv = [plsc.load_gather(ivec_v, [bbv, pidx]) for pidx in _PRM]

            @pl.loop(0, k // 16)
            def _grp(g):
                r0 = g * 16
                dvec = jnp.zeros((16,), jnp.float32)
                for j in range(16):
                    acc = None
                    for t in range(nt // 2):
                        v = rows_v[buf, r0 + j, pl.ds(32 * t, 32)]
                        e, o = plsc.unpack(v, format=plsc.PackFormat.INTERLEAVED)
                        term = e * iv[2 * t] + o * iv[2 * t + 1]
                        acc = term if acc is None else acc + term
                    dvec = jnp.where(_LANE == j, jnp.sum(acc), dvec)
                dots_v[pl.ds(r0, 16)] = dvec

            pltpu.sync_copy(dots_v, out_hbm.at[base + bb])

        @pl.loop(0, bpw // 2)
        def _pair(q):
            b0 = 2 * q
            fire(b0 + 1, 1, sem_b)
            drain(b0, 0, sem_a)
            compute(b0, 0)

            @pl.when(b0 + 2 < bpw)
            def _():
                fire(b0 + 2, 0, sem_a)

            drain(b0 + 1, 1, sem_b)
            compute(b0 + 1, 1)

    return sc_dots


def _make_reduce(b, k, bs, c, k_real, interpret=False):
    grid = (b // bs,)

    def red(dots_ref, out_ref, acc_ref):
        x = dots_ref[...]
        col = lax.broadcasted_iota(jnp.int32, (bs, k), 1)
        z = jnp.where(col < c, x, -x)
        l = jnp.log(jax.nn.sigmoid(z))
        l = jnp.where(col < k_real, l, 0.0)

        @pl.when(pl.program_id(0) == 0)
        def _():
            acc_ref[0] = 0.0

        acc_ref[0] = acc_ref[0] + jnp.sum(l)

        @pl.when(pl.program_id(0) == grid[0] - 1)
        def _():
            out_ref[0] = -acc_ref[0] / b

    return pl.pallas_call(
        red,
        grid=grid,
        in_specs=[pl.BlockSpec((bs, k), lambda i: (i, 0))],
        out_specs=pl.BlockSpec(memory_space=pltpu.SMEM),
        out_shape=jax.ShapeDtypeStruct((1,), jnp.float32),
        scratch_shapes=[pltpu.SMEM((1,), jnp.float32)],
        interpret=interpret,
    )


_sc_dots = None
_reduce = None


def kernel(iitem, oitems, nitems, W_i, W_o):
    global _sc_dots, _reduce
    if _sc_dots is None:
        _sc_dots = _make_sc_dots(B, D, K, BPW, CHUNKS)
        _reduce = _make_reduce(B, K, RED_BS, C, K_REAL)
    if oitems.dtype != jnp.int32:
        iitem = iitem.astype(jnp.int32)
        oitems = oitems.astype(jnp.int32)
        nitems = nitems.astype(jnp.int32)
    dots = _sc_dots(W_i, W_o.astype(jnp.bfloat16), oitems, nitems, iitem)
    return _reduce(dots)[0]
